# baseline (speedup 1.0000x reference)
import math
import jax
import jax.numpy as jnp
from jax.experimental import pallas as pl
from jax.experimental.pallas import tpu as pltpu

B_, L_, DM, DI, DS, DC, H_, DTR = 4, 2048, 1024, 2048, 64, 4, 16, 64
P_ = DI // H_
HN = H_ * DS
DT_MIN, DT_MAX = 1e-4, 1.0
EPS = 1e-6

Q_ = 128
NC_ = L_ // Q_
TB_ = 16
NB_ = Q_ // TB_


def _ssd_fused_kernel(z_ref, u_ref, dtE_ref, B_ref, C_ref, r_ref,
                      Af_ref, Dt_ref, cw_ref, cb_ref, nw_ref, E_ref, K_ref,
                      o_ref,
                      st, tail, ext_scr, u_scr, ubig_scr, hc_scr,
                      dA_scr, Bm_scr, y_scr):
    c = pl.program_id(1)

    @pl.when(c == 0)
    def _():
        st[...] = jnp.zeros_like(st)
        tail[...] = jnp.zeros_like(tail)

    up = u_ref[0]
    ext_scr[0:8] = tail[...]
    ext_scr[8:] = up
    tail[...] = up[Q_ - 8:]
    uc = (cb_ref[...]
          + cw_ref[0][None, :] * ext_scr[5:5 + Q_]
          + cw_ref[1][None, :] * ext_scr[6:6 + Q_]
          + cw_ref[2][None, :] * ext_scr[7:7 + Q_]
          + cw_ref[3][None, :] * ext_scr[8:8 + Q_])
    u_scr[...] = uc.reshape(NB_, TB_, P_, H_)

    dtE = dtE_ref[0]
    dA_scr[...] = jnp.exp(dtE * Af_ref[...])
    Bm_scr[...] = B_ref[0] * dtE
    DskT = Dt_ref[...]
    Emat = E_ref[...]
    Kmat = K_ref[...]

    for kb in range(NB_):
        ub3 = u_scr[kb]
        ubig_scr[...] = jnp.dot(
            ub3.reshape(TB_ * P_, H_), Emat,
            preferred_element_type=jnp.float32
        ).reshape(TB_, P_, HN)

        def body(t, carry):
            idx = kb * TB_ + t
            h = (dA_scr[idx][None, :] * st[...]
                 + Bm_scr[idx][None, :] * ubig_scr[t])
            st[...] = h
            hc_scr[t] = h * C_ref[0, idx][None, :]
            return carry

        jax.lax.fori_loop(0, TB_, body, 0, unroll=4)

        yb = jnp.dot(hc_scr[...].reshape(TB_ * P_, HN), Kmat,
                     preferred_element_type=jnp.float32)
        y_scr[kb] = yb.reshape(TB_, P_, H_) + DskT[None] * ub3

    y = y_scr[...].reshape(Q_, DI)
    zz = z_ref[0]
    g = y * (zz * jax.nn.sigmoid(zz)) + r_ref[0]
    rms = jax.lax.rsqrt(jnp.mean(g * g, axis=-1, keepdims=True) + EPS)
    o_ref[0] = g * rms * nw_ref[...]


def _ssd_fused(z, u_pre, dtE, Bp, Cp, resid, Af, DskT, conv_wT, conv_b,
               norm_w, Emat, Kmat, interpret=False):
    grid = (B_, NC_)
    blk_big = pl.BlockSpec((1, Q_, DI), lambda b, c: (b, c, 0))
    blk_bc = pl.BlockSpec((1, Q_, HN), lambda b, c: (b, c, 0))
    full2 = lambda shape: pl.BlockSpec(shape, lambda b, c: (0,) * len(shape))
    return pl.pallas_call(
        _ssd_fused_kernel,
        out_shape=jax.ShapeDtypeStruct((B_, L_, DI), jnp.float32),
        grid=grid,
        in_specs=[
            blk_big,
            blk_big,
            blk_bc,
            blk_bc,
            blk_bc,
            blk_big,
            full2((1, HN)),
            full2((P_, H_)),
            full2((DC, DI)),
            full2((1, DI)),
            full2((1, DI)),
            full2((H_, HN)),
            full2((HN, H_)),
        ],
        out_specs=blk_big,
        scratch_shapes=[
            pltpu.VMEM((P_, HN), jnp.float32),
            pltpu.VMEM((8, DI), jnp.float32),
            pltpu.VMEM((Q_ + 8, DI), jnp.float32),
            pltpu.VMEM((NB_, TB_, P_, H_), jnp.float32),
            pltpu.VMEM((TB_, P_, HN), jnp.float32),
            pltpu.VMEM((TB_, P_, HN), jnp.float32),
            pltpu.VMEM((Q_, HN), jnp.float32),
            pltpu.VMEM((Q_, HN), jnp.float32),
            pltpu.VMEM((NB_, TB_, P_, H_), jnp.float32),
        ],
        compiler_params=pltpu.CompilerParams(
            dimension_semantics=("parallel", "arbitrary"),
            vmem_limit_bytes=56 * 1024 * 1024,
        ),
        name="ssd_fused_scan",
        interpret=interpret,
    )(z, u_pre, dtE, Bp, Cp, resid, Af, DskT, conv_wT, conv_b, norm_w,
      Emat, Kmat)


def _impl(x, in_proj_w, dt_proj_w, conv_w, conv_b, A_log, Dskip, dt_bias,
          norm_weight, out_proj_w, res_proj_w, interpret=False):
    perm = (jnp.arange(H_)[None, :] * P_ + jnp.arange(P_)[:, None]).reshape(-1)
    ipw = in_proj_w
    Wz = ipw[:DI][perm]
    Wu = ipw[DI:2 * DI][perm]
    ipw_perm = jnp.concatenate([Wz, Wu, ipw[2 * DI:]], axis=0)

    p = x @ ipw_perm.T
    z = p[..., :DI]
    u_pre = p[..., DI:2 * DI]
    dt_hidden = p[..., 2 * DI:2 * DI + DTR]
    Bp = p[..., 2 * DI + DTR:2 * DI + DTR + HN]
    Cp = p[..., 2 * DI + DTR + HN:]
    dt = jnp.clip(jax.nn.softplus(dt_hidden @ dt_proj_w.T + dt_bias),
                  DT_MIN, DT_MAX)
    dtE = jnp.repeat(dt, DS, axis=-1)
    resid = (x @ res_proj_w[perm].T)
    Af = (-jnp.exp(A_log)).reshape(1, HN)
    eyeh = jnp.eye(H_, dtype=jnp.float32)
    Emat = jnp.repeat(eyeh, DS, axis=1)
    Kmat = Emat.T
    gn = _ssd_fused(z, u_pre, dtE, Bp, Cp, resid, Af, Dskip.T,
                    conv_w[perm].T, conv_b[perm].reshape(1, DI),
                    norm_weight[perm].reshape(1, DI), Emat, Kmat,
                    interpret=interpret)
    return gn @ out_proj_w[:, perm].T


def kernel(x, in_proj_w, dt_proj_w, conv_w, conv_b, A_log, Dskip, dt_bias,
           norm_weight, out_proj_w, res_proj_w):
    return _impl(x, in_proj_w, dt_proj_w, conv_w, conv_b, A_log, Dskip,
                 dt_bias, norm_weight, out_proj_w, res_proj_w)
